# baseline (speedup 1.0000x reference)
import jax
import jax.numpy as jnp
from jax.experimental import pallas as pl
from jax.experimental.pallas import tpu as pltpu

_BATCH = 1
_SEQ = 8192
_HEADS = 32
_HEAD_DIM = 128
_Q = 16
_ROW = _HEADS * _HEAD_DIM

_BS = 128


def _copy_scatter_body(pos_ref, kc_ref, vc_ref, kv_ref, vv_ref, ko_ref, vo_ref):
    i = pl.program_id(0)
    ko_ref[...] = kc_ref[...]
    vo_ref[...] = vc_ref[...]
    base = i * _BS

    def body(j, carry):
        p = pos_ref[j]
        local = p - base

        @pl.when(jnp.logical_and(p >= base, p < base + _BS))
        def _():
            ko_ref[pl.ds(local, 1), :] = kv_ref[pl.ds(j, 1), :]
            vo_ref[pl.ds(local, 1), :] = vv_ref[pl.ds(j, 1), :]

        return carry

    jax.lax.fori_loop(0, _Q, body, 0)


def kernel(k_cache, v_cache, input_pos, k_val, v_val):
    kc = k_cache.reshape(_SEQ, _ROW)
    vc = v_cache.reshape(_SEQ, _ROW)
    kv = k_val.reshape(_Q, _ROW)
    vv = v_val.reshape(_Q, _ROW)
    pos = input_pos.astype(jnp.int32)

    grid = (_SEQ // _BS,)
    out_k, out_v = pl.pallas_call(
        _copy_scatter_body,
        grid_spec=pltpu.PrefetchScalarGridSpec(
            num_scalar_prefetch=1,
            grid=grid,
            in_specs=[
                pl.BlockSpec((_BS, _ROW), lambda i, pos: (i, 0)),
                pl.BlockSpec((_BS, _ROW), lambda i, pos: (i, 0)),
                pl.BlockSpec((_Q, _ROW), lambda i, pos: (0, 0)),
                pl.BlockSpec((_Q, _ROW), lambda i, pos: (0, 0)),
            ],
            out_specs=[
                pl.BlockSpec((_BS, _ROW), lambda i, pos: (i, 0)),
                pl.BlockSpec((_BS, _ROW), lambda i, pos: (i, 0)),
            ],
        ),
        out_shape=[
            jax.ShapeDtypeStruct((_SEQ, _ROW), jnp.float32),
            jax.ShapeDtypeStruct((_SEQ, _ROW), jnp.float32),
        ],
    )(pos, kc, vc, kv, vv)

    return (
        out_k.reshape(_BATCH, _SEQ, _HEADS, _HEAD_DIM),
        out_v.reshape(_BATCH, _SEQ, _HEADS, _HEAD_DIM),
    )
